# baseline (speedup 1.0000x reference)
import functools

import jax
import jax.numpy as jnp
from jax import lax
from jax.experimental import pallas as pl
from jax.experimental.pallas import tpu as pltpu

N_TOKENS = 32768
EMB_DIM = 768
NUM_EXPERTS = 64
BEST_K = 8
BLOCK = 2048

_NEG_INF = float("-inf")


def _router_body(x_ref, w_ref, b_ref, router_ref, idx_ref):
    x_blk = x_ref[...]
    w = w_ref[...]
    logits = lax.dot_general(
        w, x_blk, (((1,), (1,)), ((), ())),
        preferred_element_type=jnp.float32,
    )
    logits = logits + b_ref[...]

    rowf = lax.broadcasted_iota(jnp.int32, logits.shape, 0).astype(jnp.float32)
    masked = logits
    vals = []
    idxs = []
    for _ in range(BEST_K):
        m = jnp.max(masked, axis=0, keepdims=True)
        idxf = jnp.min(
            jnp.where(masked == m, rowf, float(NUM_EXPERTS)),
            axis=0,
            keepdims=True,
        )
        onehot = rowf == idxf
        masked = jnp.where(onehot, _NEG_INF, masked)
        vals.append(m)
        idxs.append(idxf)

    top1 = vals[0]
    e = jnp.where(masked == _NEG_INF, jnp.exp(logits - top1), 0.0)
    denom = jnp.sum(e, axis=0, keepdims=True)
    router_ref[...] = e / denom
    idxf8 = jnp.concatenate(idxs, axis=0)
    idx_ref[...] = idxf8.astype(jnp.int32)


@jax.jit
def kernel(x, W, b):
    b2 = b.reshape(NUM_EXPERTS, 1)
    grid = (N_TOKENS // BLOCK,)
    router, idxs = pl.pallas_call(
        _router_body,
        grid=grid,
        in_specs=[
            pl.BlockSpec((BLOCK, EMB_DIM), lambda i: (i, 0)),
            pl.BlockSpec((NUM_EXPERTS, EMB_DIM), lambda i: (0, 0)),
            pl.BlockSpec((NUM_EXPERTS, 1), lambda i: (0, 0)),
        ],
        out_specs=[
            pl.BlockSpec((NUM_EXPERTS, BLOCK), lambda i: (0, i)),
            pl.BlockSpec((BEST_K, BLOCK), lambda i: (0, i)),
        ],
        out_shape=[
            jax.ShapeDtypeStruct((NUM_EXPERTS, N_TOKENS), jnp.float32),
            jax.ShapeDtypeStruct((BEST_K, N_TOKENS), jnp.int32),
        ],
        compiler_params=pltpu.CompilerParams(
            dimension_semantics=("arbitrary",),
        ),
    )(x, W, b2)
    return (router.T, idxs.T)


# speedup vs baseline: 1.1071x; 1.1071x over previous
import jax
import jax.numpy as jnp
from jax import lax
from jax.experimental import pallas as pl
from jax.experimental.pallas import tpu as pltpu

N_TOKENS = 32768
EMB_DIM = 768
NUM_EXPERTS = 64
BEST_K = 8
BLOCK = 4096

_NEG_INF = float("-inf")


def _router_body(x_ref, w_ref, b_ref, router_ref, idx_ref):
    x_blk = x_ref[...]
    w = w_ref[...]
    logits = lax.dot_general(
        w, x_blk, (((1,), (1,)), ((), ())),
        preferred_element_type=jnp.float32,
    )
    logits = logits + b_ref[...]

    rowf = lax.broadcasted_iota(jnp.int32, logits.shape, 0).astype(jnp.float32)
    masked = logits
    vals = []
    idxs = []
    for _ in range(BEST_K):
        m = jnp.max(masked, axis=0, keepdims=True)
        idxf = jnp.min(
            jnp.where(masked == m, rowf, float(NUM_EXPERTS)),
            axis=0,
            keepdims=True,
        )
        onehot = rowf == idxf
        masked = jnp.where(onehot, _NEG_INF, masked)
        vals.append(m)
        idxs.append(idxf)

    top1 = vals[0]
    e = jnp.where(masked == _NEG_INF, jnp.exp(logits - top1), 0.0)
    denom = jnp.sum(e, axis=0, keepdims=True)
    router_ref[...] = e / denom
    idxf8 = jnp.concatenate(idxs, axis=0)
    idx_ref[...] = idxf8.astype(jnp.int32)


@jax.jit
def kernel(x, W, b):
    b2 = b.reshape(NUM_EXPERTS, 1)
    grid = (N_TOKENS // BLOCK,)
    router, idxs = pl.pallas_call(
        _router_body,
        grid=grid,
        in_specs=[
            pl.BlockSpec((BLOCK, EMB_DIM), lambda i: (i, 0)),
            pl.BlockSpec((NUM_EXPERTS, EMB_DIM), lambda i: (0, 0)),
            pl.BlockSpec((NUM_EXPERTS, 1), lambda i: (0, 0)),
        ],
        out_specs=[
            pl.BlockSpec((NUM_EXPERTS, BLOCK), lambda i: (0, i)),
            pl.BlockSpec((BEST_K, BLOCK), lambda i: (0, i)),
        ],
        out_shape=[
            jax.ShapeDtypeStruct((NUM_EXPERTS, N_TOKENS), jnp.float32),
            jax.ShapeDtypeStruct((BEST_K, N_TOKENS), jnp.int32),
        ],
        compiler_params=pltpu.CompilerParams(
            dimension_semantics=("arbitrary",),
        ),
    )(x, W, b2)
    return (router.T, idxs.T)
